# baseline (speedup 1.0000x reference)
import jax
import jax.numpy as jnp
from jax.experimental import pallas as pl
from jax.experimental.pallas import tpu as pltpu

_D = 4096
_T = 50
_HB, _WB, _BS = 15, 20, 8
_NB = _HB * _WB
_DT = 512
_NT = _D // _DT

_INV_PI = 0.3183098861837907
_PI_HI = 3.140625
_PI_LO = 9.67653589793e-4
_C3 = -1.66666571e-01
_C5 = 8.33301756e-03
_C7 = -1.98066341e-04
_C9 = 2.60009581e-06


def _fast_sin(x):
    f32 = jnp.float32
    t = x * f32(_INV_PI)
    k = jax.lax.round(t, jax.lax.RoundingMethod.TO_NEAREST_EVEN)
    par = k.astype(jnp.int32) & jnp.int32(1)
    r = (x - k * f32(_PI_HI)) - k * f32(_PI_LO)
    rb = jax.lax.bitcast_convert_type(r, jnp.int32) ^ (par << 31)
    r = jax.lax.bitcast_convert_type(rb, jnp.float32)
    r2 = r * r
    q = ((f32(_C9) * r2 + f32(_C7)) * r2 + f32(_C5)) * r2 + f32(_C3)
    return r + r * r2 * q


def _body(xr_ref, w_ref, pos_ref, time_ref, pol_ref, out_ref):
    w = w_ref[...]
    pos = pos_ref[...]
    pw0 = pos * (w * pol_ref[0:1, :])
    pw1 = pos * (w * pol_ref[1:2, :])
    pw = jnp.concatenate([pw0, pw1], axis=0)
    ones_row = jnp.ones((1, 2 * _NB), jnp.float32)
    tmax = time_ref.shape[0] - 1

    def step(t, acc):
        xt = xr_ref[t]
        c = jnp.sum(xt.reshape(2 * _NB, 64), axis=1, keepdims=True)
        u = _fast_sin(c * pw)
        red = jnp.dot(ones_row, u, preferred_element_type=jnp.float32)
        tt = jnp.minimum(t, tmax)
        return acc + red * time_ref[pl.ds(tt, 1), :]

    acc = jax.lax.fori_loop(0, _T, step, jnp.zeros((1, _DT), jnp.float32))
    out_ref[...] = jnp.where(acc > 0.0, 1.0, -1.0)


def kernel(x, proj_w, position, time, polarity):
    Tn = x.shape[0]
    xr = (
        x.reshape(Tn, 2, _HB, _BS, _WB, _BS)
        .transpose(0, 1, 2, 4, 3, 5)
        .reshape(Tn, 2, _NB, _BS * _BS)
    )
    w_row = proj_w.reshape(1, _D)
    out = pl.pallas_call(
        _body,
        grid=(_NT,),
        in_specs=[
            pl.BlockSpec((Tn, 2, _NB, _BS * _BS), lambda j: (0, 0, 0, 0)),
            pl.BlockSpec((1, _DT), lambda j: (0, j)),
            pl.BlockSpec((_NB, _DT), lambda j: (0, j)),
            pl.BlockSpec((time.shape[0], _DT), lambda j: (0, j)),
            pl.BlockSpec((2, _DT), lambda j: (0, j)),
        ],
        out_specs=pl.BlockSpec((1, _DT), lambda j: (0, j)),
        out_shape=jax.ShapeDtypeStruct((1, _D), jnp.float32),
        compiler_params=pltpu.CompilerParams(
            dimension_semantics=("parallel",)
        ),
    )(xr, w_row, position, time, polarity)
    return out.reshape(_D)


# speedup vs baseline: 1.2131x; 1.2131x over previous
import jax
import jax.numpy as jnp
from jax.experimental import pallas as pl
from jax.experimental.pallas import tpu as pltpu

_D = 4096
_T = 50
_HB, _WB, _BS = 15, 20, 8
_NB = _HB * _WB
_DT = 512
_NT = _D // _DT

_INV_PI = 0.3183098861837907
_PI_HI = 3.140625
_PI_LO = 9.67653589793e-4
_C3 = -0.16665682
_C5 = 8.31238e-03
_C7 = -1.8493e-04


def _fast_sin(x):
    f32 = jnp.float32
    t = x * f32(_INV_PI)
    k = jax.lax.round(t, jax.lax.RoundingMethod.TO_NEAREST_EVEN)
    ki = k.astype(jnp.int32)
    r = (x - k * f32(_PI_HI)) - k * f32(_PI_LO)
    rb = jax.lax.bitcast_convert_type(r, jnp.int32) ^ (ki << 31)
    r = jax.lax.bitcast_convert_type(rb, jnp.float32)
    r2 = r * r
    q = (f32(_C7) * r2 + f32(_C5)) * r2 + f32(_C3)
    return r + r * r2 * q


def _body(xr_ref, w_ref, pos_ref, time_ref, pol_ref, out_ref):
    w = w_ref[...]
    pos = pos_ref[...]
    pw0 = pos * (w * pol_ref[0:1, :])
    pw1 = pos * (w * pol_ref[1:2, :])
    pw = jnp.concatenate([pw0, pw1], axis=0)
    tmax = time_ref.shape[0] - 1

    def step(t, acc):
        xt = xr_ref[t]
        c = jnp.sum(xt.reshape(2 * _NB, 64), axis=1, keepdims=True)
        u = _fast_sin(c * pw)
        red = jnp.sum(u, axis=0, keepdims=True)
        tt = jnp.minimum(t, tmax)
        return acc + red * time_ref[pl.ds(tt, 1), :]

    acc = jax.lax.fori_loop(0, _T, step, jnp.zeros((1, _DT), jnp.float32))
    out_ref[...] = jnp.where(acc > 0.0, 1.0, -1.0)


def kernel(x, proj_w, position, time, polarity):
    Tn = x.shape[0]
    xr = (
        x.reshape(Tn, 2, _HB, _BS, _WB, _BS)
        .transpose(0, 1, 2, 4, 3, 5)
        .reshape(Tn, 2, _NB, _BS * _BS)
    )
    w_row = proj_w.reshape(1, _D)
    out = pl.pallas_call(
        _body,
        grid=(_NT,),
        in_specs=[
            pl.BlockSpec((Tn, 2, _NB, _BS * _BS), lambda j: (0, 0, 0, 0)),
            pl.BlockSpec((1, _DT), lambda j: (0, j)),
            pl.BlockSpec((_NB, _DT), lambda j: (0, j)),
            pl.BlockSpec((time.shape[0], _DT), lambda j: (0, j)),
            pl.BlockSpec((2, _DT), lambda j: (0, j)),
        ],
        out_specs=pl.BlockSpec((1, _DT), lambda j: (0, j)),
        out_shape=jax.ShapeDtypeStruct((1, _D), jnp.float32),
        compiler_params=pltpu.CompilerParams(
            dimension_semantics=("parallel",)
        ),
    )(xr, w_row, position, time, polarity)
    return out.reshape(_D)
